# baseline (speedup 1.0000x reference)
import functools

import jax
import jax.numpy as jnp
from jax import lax
from jax.experimental import pallas as pl
from jax.experimental.pallas import tpu as pltpu
from jax.experimental.pallas import tpu_sc as plsc

_NC = 2
_NS = 16
_NW = _NC * _NS
_LANES = 16

_C1 = 16
_C2 = 32


def _sc_body(B, S, L, D, led_hbm, ids_hbm, pos_hbm, struct_hbm, out_hbm,
             led_v, gat_v, idx_v, pos1_v, pos2_v, sem):
    wid = lax.axis_index("s") * _NC + lax.axis_index("c")

    rows1 = L // _NW
    rows2 = (S - L) // _NW
    nslice = D // _LANES

    base1 = wid * rows1
    base2 = L + wid * rows2

    pltpu.sync_copy(pos_hbm.at[pl.ds(base1, rows1)], pos1_v)
    pltpu.sync_copy(pos_hbm.at[pl.ds(base2, rows2)], pos2_v)
    for b in range(B):
        pltpu.sync_copy(ids_hbm.at[pl.ds(b * L + base1, rows1)], idx_v.at[b])

    for chunk in range(rows1 // _C1):
        s0 = base1 + chunk * _C1
        pltpu.async_copy(
            led_hbm.at[pos1_v.at[pl.ds(chunk * _C1, _C1)]],
            led_v.at[pl.ds(0, _C1)], sem).wait()
        for b in range(B):
            pltpu.async_copy(
                struct_hbm.at[idx_v.at[b, pl.ds(chunk * _C1, _C1)]],
                gat_v.at[b], sem).wait()

        def add_row(r, _):
            for c in range(nslice):
                led_slice = led_v[r, pl.ds(c * _LANES, _LANES)]
                for b in range(B):
                    gat_v[b, r, pl.ds(c * _LANES, _LANES)] += led_slice
            return 0

        lax.fori_loop(0, _C1, add_row, 0)
        for b in range(B):
            pltpu.sync_copy(gat_v.at[b], out_hbm.at[pl.ds(b * S + s0, _C1)])

    for chunk in range(rows2 // _C2):
        s0 = base2 + chunk * _C2
        pltpu.async_copy(
            led_hbm.at[pos2_v.at[pl.ds(chunk * _C2, _C2)]], led_v, sem).wait()
        for b in range(B):
            pltpu.sync_copy(led_v, out_hbm.at[pl.ds(b * S + s0, _C2)])


def _build_sc_call(B, S, L, D):
    mesh = plsc.VectorSubcoreMesh(core_axis_name="c", subcore_axis_name="s")
    body = functools.partial(_sc_body, B, S, L, D)
    rows1 = L // _NW
    rows2 = (S - L) // _NW
    return pl.kernel(
        body,
        out_type=jax.ShapeDtypeStruct((B * S, D), jnp.float32),
        mesh=mesh,
        scratch_types=[
            pltpu.VMEM((_C2, D), jnp.float32),
            pltpu.VMEM((B, _C1, D), jnp.float32),
            pltpu.VMEM((B, rows1), jnp.int32),
            pltpu.VMEM((rows1,), jnp.int32),
            pltpu.VMEM((rows2,), jnp.int32),
            pltpu.SemaphoreType.DMA,
        ],
        name="led_struct_pos_emb_sc",
    )


def kernel(led_pos_table, struct_table, struct_position_ids, batch, seq_len,
           past_key_values_length):
    S, D = led_pos_table.shape
    B, L = struct_position_ids.shape

    offset = past_key_values_length + (seq_len - S) + (batch - B)
    positions = jnp.clip(offset + jnp.arange(S, dtype=jnp.int32), 0, S - 1)
    positions = positions.astype(jnp.int32)

    ids_flat = struct_position_ids.reshape(B * L)
    out2d = _build_sc_call(B, S, L, D)(
        led_pos_table, ids_flat, positions, struct_table)
    return out2d.reshape(B, S, D)
